# baseline (speedup 1.0000x reference)
import functools

import jax
import jax.numpy as jnp
from jax import lax
from jax.experimental import pallas as pl
from jax.experimental.pallas import tpu as pltpu
from jax.experimental.pallas import tpu_sc as plsc

B = 1024
S = 200
HID = 128
VOCAB = 100000
TYPE_V = 6
L = 16
NW = 32
TOK_PER_W = B * S // NW
CHUNK = 160
NCHUNK = TOK_PER_W // CHUNK
IDXW = 20
NJ = CHUNK // IDXW
NGC = CHUNK // L

_mesh = plsc.VectorSubcoreMesh(core_axis_name="c", subcore_axis_name="s")

_f32 = jnp.float32
_i32 = jnp.int32


@functools.partial(
    pl.kernel,
    out_type=jax.ShapeDtypeStruct((B * S, HID), _f32),
    mesh=_mesh,
    compiler_params=pltpu.CompilerParams(needs_layout_passes=False),
    scratch_types=[
        [pltpu.VMEM((NJ, IDXW), _i32)] * 2,
        [pltpu.VMEM((CHUNK,), _i32)] * 2,
        [pltpu.VMEM((CHUNK,), _i32)] * 2,
        [pltpu.VMEM((CHUNK, HID), _f32)] * 2,
        pltpu.VMEM((S, HID), _f32),
        pltpu.VMEM((TYPE_V * HID,), _f32),
        pltpu.VMEM((HID,), _f32),
        pltpu.VMEM((HID,), _f32),
        [pltpu.SemaphoreType.DMA] * 2,
        [pltpu.SemaphoreType.DMA] * 2,
        [pltpu.SemaphoreType.DMA] * 2,
    ],
)
def _emb_ln(tok2d_hbm, tok1d_hbm, W_word_hbm, posseg_hbm, typef_hbm,
            gamma_hbm, beta_hbm, out_hbm, idx_v, tok_v, t128_v, rows_v,
            posseg_v, type_v, gamma_v, beta_v, isem, gsem, osem):
    wid = lax.axis_index("s") * 2 + lax.axis_index("c")
    pltpu.sync_copy(posseg_hbm, posseg_v)
    pltpu.sync_copy(typef_hbm, type_v)
    pltpu.sync_copy(gamma_hbm, gamma_v)
    pltpu.sync_copy(beta_hbm, beta_v)

    def drain(src, dst, sem):
        pltpu.make_async_copy(src, dst, sem).wait()

    def fire_idx(c, p):
        tok0 = wid * TOK_PER_W + c * CHUNK
        r0 = pl.multiple_of(tok0 // IDXW, 8)
        pltpu.async_copy(tok2d_hbm.at[pl.ds(r0, NJ)], idx_v[p], isem[p])
        pltpu.async_copy(tok1d_hbm.at[pl.ds(tok0, CHUNK)], tok_v[p], isem[p])

    def fire_gathers(c, p):
        drain(tok2d_hbm.at[pl.ds(0, NJ)], idx_v[p], isem[p])
        drain(tok1d_hbm.at[pl.ds(0, CHUNK)], tok_v[p], isem[p])
        for k in range(NGC):
            tv = tok_v[p][pl.ds(L * k, L)]
            t128_v[p][pl.ds(L * k, L)] = lax.rem(tv, TYPE_V) * HID
        for j in range(NJ):
            pltpu.async_copy(W_word_hbm.at[idx_v[p].at[j]],
                             rows_v[p].at[pl.ds(IDXW * j, IDXW)], gsem[p])

    def compute(c, p):
        drain(out_hbm.at[pl.ds(0, CHUNK)], rows_v[p], gsem[p])
        cols = [jnp.arange(L * j, L * j + L, dtype=_i32) for j in range(HID // L)]

        @pl.loop(0, NGC)
        def _grp(g):
            m0 = c * CHUNK + L * g
            for t in range(L):
                i = L * g + t
                s_i = lax.rem(m0 + t, S)
                tb = plsc.load_gather(t128_v[p], [jnp.full((L,), i, _i32)])
                accs = []
                for j in range(HID // L):
                    w = rows_v[p][i, pl.ds(L * j, L)]
                    pv = posseg_v[s_i, pl.ds(L * j, L)]
                    ty = plsc.load_gather(type_v, [tb + cols[j]])
                    accs.append(w + pv + ty)
                tot = accs[0]
                for j in range(1, HID // L):
                    tot = tot + accs[j]
                total = jnp.sum(tot)
                sq = accs[0] * accs[0]
                for j in range(1, HID // L):
                    sq = sq + accs[j] * accs[j]
                totalsq = jnp.sum(sq)
                mean = total * (1.0 / HID)
                var = totalsq * (1.0 / HID) - mean * mean
                rv = jnp.full((L,), var + 1e-12, dtype=_f32)
                bi = plsc.bitcast(rv, _i32)
                bi = 0x5F3759DF - lax.shift_right_logical(bi, 1)
                y = plsc.bitcast(bi, _f32)
                for _ in range(2):
                    y = y * (1.5 - 0.5 * rv * y * y)
                for j in range(HID // L):
                    gm = gamma_v[pl.ds(L * j, L)]
                    be = beta_v[pl.ds(L * j, L)]
                    rows_v[p][i, pl.ds(L * j, L)] = (accs[j] - mean) * y * gm + be

        tok0 = wid * TOK_PER_W + c * CHUNK
        pltpu.async_copy(rows_v[p], out_hbm.at[pl.ds(tok0, CHUNK)], osem[p])

    def drain_out(p):
        drain(out_hbm.at[pl.ds(0, CHUNK)], rows_v[p], osem[p])

    fire_idx(0, 0)
    fire_gathers(0, 0)
    fire_idx(1, 1)
    fire_gathers(1, 1)

    @pl.loop(0, NCHUNK, step=2)
    def _body(c):
        compute(c, 0)
        fire_idx(lax.rem(c + 2, NCHUNK), 0)
        compute(c + 1, 1)
        fire_idx(lax.rem(c + 3, NCHUNK), 1)
        drain_out(0)
        fire_gathers(lax.rem(c + 2, NCHUNK), 0)
        drain_out(1)
        fire_gathers(lax.rem(c + 3, NCHUNK), 1)

    drain(out_hbm.at[pl.ds(0, CHUNK)], rows_v[0], gsem[0])
    drain(out_hbm.at[pl.ds(0, CHUNK)], rows_v[1], gsem[1])


def kernel(token_ids, segment_ids, W_word, W_pos, W_seg, W_type, gamma, beta):
    del segment_ids
    posseg = W_pos[:S] + W_seg[0][None, :]
    typef = W_type.reshape(TYPE_V * HID)
    tok1d = token_ids.reshape(B * S)
    tok2d = tok1d.reshape(B * S // IDXW, IDXW)
    out = _emb_ln(tok2d, tok1d, W_word, posseg, typef, gamma, beta)
    return out.reshape(B, S, HID)


# speedup vs baseline: 3.5991x; 3.5991x over previous
import functools

import jax
import jax.numpy as jnp
from jax import lax
from jax.experimental import pallas as pl
from jax.experimental.pallas import tpu as pltpu
from jax.experimental.pallas import tpu_sc as plsc

B = 1024
S = 200
HID = 128
VOCAB = 100000
TYPE_V = 6
L = 16
NW = 32
TOK_PER_W = B * S // NW
CHUNK = 160
NCHUNK = TOK_PER_W // CHUNK
IDXW = 80
NJ = CHUNK // IDXW
KPR = IDXW // L
NGC = CHUNK // L
PERIOD = 400 // L

_mesh = plsc.VectorSubcoreMesh(core_axis_name="c", subcore_axis_name="s")

_f32 = jnp.float32
_i32 = jnp.int32


@functools.partial(
    pl.kernel,
    out_type=jax.ShapeDtypeStruct((B * S, HID), _f32),
    mesh=_mesh,
    compiler_params=pltpu.CompilerParams(needs_layout_passes=False),
    scratch_types=[
        [pltpu.VMEM((NJ, IDXW), _i32)] * 2,
        [pltpu.VMEM((CHUNK,), _i32)] * 2,
        [pltpu.VMEM((NJ, IDXW), _i32)] * 2,
        [pltpu.VMEM((CHUNK, HID), _f32)] * 2,
        [pltpu.VMEM((CHUNK, HID), _f32)] * 2,
        pltpu.VMEM((PERIOD, L), _i32),
        pltpu.VMEM((HID,), _f32),
        pltpu.VMEM((HID,), _f32),
        [pltpu.SemaphoreType.DMA] * 2,
        [pltpu.SemaphoreType.DMA] * 2,
        [pltpu.SemaphoreType.DMA] * 2,
    ],
)
def _emb_ln(tok1d_hbm, W_word_hbm, comb_hbm, pos6_hbm, gamma_hbm,
            beta_hbm, out_hbm, idx_v, tok_v, idx2_v, rows_v, add_v, pos6_v,
            gamma_v, beta_v, isem, gsem, osem):
    wid = lax.axis_index("s") * 2 + lax.axis_index("c")
    pltpu.sync_copy(pos6_hbm, pos6_v)
    pltpu.sync_copy(gamma_hbm, gamma_v)
    pltpu.sync_copy(beta_hbm, beta_v)

    def drain(src, dst, sem):
        pltpu.make_async_copy(src, dst, sem).wait()

    def fire_idx(c, p):
        tok0 = wid * TOK_PER_W + c * CHUNK
        pltpu.async_copy(tok1d_hbm.at[pl.ds(tok0, CHUNK)], tok_v[p], isem[p])

    def fire_gathers(c, p):
        drain(tok1d_hbm.at[pl.ds(0, CHUNK)], tok_v[p], isem[p])
        for r in range(NJ):
            for k in range(KPR):
                g = lax.rem(c * NGC + KPR * r + k, PERIOD)
                tv = tok_v[p][pl.ds(IDXW * r + L * k, L)]
                idx_v[p][r, pl.ds(L * k, L)] = tv
                idx2_v[p][r, pl.ds(L * k, L)] = pos6_v[g] + lax.rem(tv, TYPE_V)
        for r in range(NJ):
            pltpu.async_copy(W_word_hbm.at[idx_v[p].at[r]],
                             rows_v[p].at[pl.ds(IDXW * r, IDXW)], gsem[p])
            pltpu.async_copy(comb_hbm.at[idx2_v[p].at[r]],
                             add_v[p].at[pl.ds(IDXW * r, IDXW)], gsem[p])

    def compute(c, p):
        drain(out_hbm.at[pl.ds(0, CHUNK)], rows_v[p], gsem[p])
        drain(out_hbm.at[pl.ds(0, CHUNK)], add_v[p], gsem[p])

        @pl.loop(0, NGC)
        def _grp(g):
            for t in range(L):
                i = L * g + t
                accs = []
                for j in range(HID // L):
                    w = rows_v[p][i, pl.ds(L * j, L)]
                    a = add_v[p][i, pl.ds(L * j, L)]
                    accs.append(w + a)
                tot = accs[0]
                for j in range(1, HID // L):
                    tot = tot + accs[j]
                total = jnp.sum(tot)
                sq = accs[0] * accs[0]
                for j in range(1, HID // L):
                    sq = sq + accs[j] * accs[j]
                totalsq = jnp.sum(sq)
                mean = total * (1.0 / HID)
                var = totalsq * (1.0 / HID) - mean * mean
                rv = jnp.full((L,), var + 1e-12, dtype=_f32)
                bi = plsc.bitcast(rv, _i32)
                bi = 0x5F3759DF - lax.shift_right_logical(bi, 1)
                y = plsc.bitcast(bi, _f32)
                for _ in range(2):
                    y = y * (1.5 - 0.5 * rv * y * y)
                for j in range(HID // L):
                    gm = gamma_v[pl.ds(L * j, L)]
                    be = beta_v[pl.ds(L * j, L)]
                    rows_v[p][i, pl.ds(L * j, L)] = (accs[j] - mean) * y * gm + be

        tok0 = wid * TOK_PER_W + c * CHUNK
        pltpu.async_copy(rows_v[p], out_hbm.at[pl.ds(tok0, CHUNK)], osem[p])

    def drain_out(p):
        drain(out_hbm.at[pl.ds(0, CHUNK)], rows_v[p], osem[p])

    fire_idx(0, 0)
    fire_gathers(0, 0)
    fire_idx(1, 1)
    fire_gathers(1, 1)

    @pl.loop(0, NCHUNK, step=2)
    def _body(c):
        compute(c, 0)
        fire_idx(lax.rem(c + 2, NCHUNK), 0)
        compute(c + 1, 1)
        fire_idx(lax.rem(c + 3, NCHUNK), 1)
        drain_out(0)
        fire_gathers(lax.rem(c + 2, NCHUNK), 0)
        drain_out(1)
        fire_gathers(lax.rem(c + 3, NCHUNK), 1)

    drain(out_hbm.at[pl.ds(0, CHUNK)], rows_v[0], gsem[0])
    drain(out_hbm.at[pl.ds(0, CHUNK)], add_v[0], gsem[0])
    drain(out_hbm.at[pl.ds(0, CHUNK)], rows_v[1], gsem[1])
    drain(out_hbm.at[pl.ds(0, CHUNK)], add_v[1], gsem[1])


def kernel(token_ids, segment_ids, W_word, W_pos, W_seg, W_type, gamma, beta):
    del segment_ids
    comb = (W_pos[:S, None, :] + W_seg[0][None, None, :]
            + W_type[None, :, :]).reshape(S * TYPE_V, HID)
    pos6 = (TYPE_V * (jnp.arange(PERIOD * L, dtype=_i32) % S)).reshape(PERIOD, L)
    tok1d = token_ids.reshape(B * S)
    out = _emb_ln(tok1d, W_word, comb, pos6, gamma, beta)
    return out.reshape(B, S, HID)


# speedup vs baseline: 3.5999x; 1.0002x over previous
import functools

import jax
import jax.numpy as jnp
from jax import lax
from jax.experimental import pallas as pl
from jax.experimental.pallas import tpu as pltpu
from jax.experimental.pallas import tpu_sc as plsc

B = 1024
S = 200
HID = 128
VOCAB = 100000
TYPE_V = 6
L = 16
NW = 32
TOK_PER_W = B * S // NW
CHUNK = 160
NCHUNK = TOK_PER_W // CHUNK
IDXW = 80
NJ = CHUNK // IDXW
KPR = IDXW // L
NGC = CHUNK // L
PERIOD = 400 // L

_mesh = plsc.VectorSubcoreMesh(core_axis_name="c", subcore_axis_name="s")

_f32 = jnp.float32
_i32 = jnp.int32


@functools.partial(
    pl.kernel,
    out_type=jax.ShapeDtypeStruct((B * S, HID), _f32),
    mesh=_mesh,
    compiler_params=pltpu.CompilerParams(needs_layout_passes=False),
    scratch_types=[
        [pltpu.VMEM((NJ, IDXW), _i32)] * 2,
        [pltpu.VMEM((CHUNK,), _i32)] * 2,
        [pltpu.VMEM((NJ, IDXW), _i32)] * 2,
        [pltpu.VMEM((CHUNK, HID), _f32)] * 2,
        [pltpu.VMEM((CHUNK, HID), _f32)] * 2,
        pltpu.VMEM((PERIOD, L), _i32),
        pltpu.VMEM((HID,), _f32),
        pltpu.VMEM((HID,), _f32),
        [pltpu.SemaphoreType.DMA] * 2,
        [pltpu.SemaphoreType.DMA] * 2,
        [pltpu.SemaphoreType.DMA] * 2,
    ],
)
def _emb_ln(tok1d_hbm, W_word_hbm, comb_hbm, pos6_hbm, gamma_hbm,
            beta_hbm, out_hbm, idx_v, tok_v, idx2_v, rows_v, add_v, pos6_v,
            gamma_v, beta_v, isem, gsem, osem):
    wid = lax.axis_index("s") * 2 + lax.axis_index("c")
    pltpu.sync_copy(pos6_hbm, pos6_v)
    pltpu.sync_copy(gamma_hbm, gamma_v)
    pltpu.sync_copy(beta_hbm, beta_v)

    def drain(src, dst, sem):
        pltpu.make_async_copy(src, dst, sem).wait()

    def fire_idx(c, p):
        tok0 = wid * TOK_PER_W + c * CHUNK
        pltpu.async_copy(tok1d_hbm.at[pl.ds(tok0, CHUNK)], tok_v[p], isem[p])

    def fire_gathers(c, p):
        drain(tok1d_hbm.at[pl.ds(0, CHUNK)], tok_v[p], isem[p])
        for r in range(NJ):
            for k in range(KPR):
                g = lax.rem(c * NGC + KPR * r + k, PERIOD)
                tv = tok_v[p][pl.ds(IDXW * r + L * k, L)]
                idx_v[p][r, pl.ds(L * k, L)] = tv
                idx2_v[p][r, pl.ds(L * k, L)] = pos6_v[g] + lax.rem(tv, TYPE_V)
        for r in range(NJ):
            pltpu.async_copy(W_word_hbm.at[idx_v[p].at[r]],
                             rows_v[p].at[pl.ds(IDXW * r, IDXW)], gsem[p])
            pltpu.async_copy(comb_hbm.at[idx2_v[p].at[r]],
                             add_v[p].at[pl.ds(IDXW * r, IDXW)], gsem[p])

    def compute(c, p):
        drain(out_hbm.at[pl.ds(0, CHUNK)], rows_v[p], gsem[p])
        drain(out_hbm.at[pl.ds(0, CHUNK)], add_v[p], gsem[p])

        @pl.loop(0, NGC)
        def _grp(g):
            for t in range(L):
                i = L * g + t
                accs = []
                for j in range(HID // L):
                    w = rows_v[p][i, pl.ds(L * j, L)]
                    a = add_v[p][i, pl.ds(L * j, L)]
                    accs.append(w + a)
                tot = accs[0]
                for j in range(1, HID // L):
                    tot = tot + accs[j]
                total = jnp.sum(tot)
                sq = accs[0] * accs[0]
                for j in range(1, HID // L):
                    sq = sq + accs[j] * accs[j]
                totalsq = jnp.sum(sq)
                mean = total * (1.0 / HID)
                var = totalsq * (1.0 / HID) - mean * mean
                rv = var + 1e-12
                bi = lax.bitcast_convert_type(rv, _i32)
                bi = 0x5F3759DF - lax.shift_right_logical(bi, 1)
                y = lax.bitcast_convert_type(bi, _f32)
                h = 0.5 * rv
                y = y * (1.5 - h * y * y)
                y = y * (1.5 - h * y * y)
                ym = jnp.full((L,), y, dtype=_f32)
                mb = jnp.full((L,), mean, dtype=_f32)
                for j in range(HID // L):
                    gm = gamma_v[pl.ds(L * j, L)]
                    be = beta_v[pl.ds(L * j, L)]
                    rows_v[p][i, pl.ds(L * j, L)] = (accs[j] - mb) * ym * gm + be

        tok0 = wid * TOK_PER_W + c * CHUNK
        pltpu.async_copy(rows_v[p], out_hbm.at[pl.ds(tok0, CHUNK)], osem[p])

    def drain_out(p):
        drain(out_hbm.at[pl.ds(0, CHUNK)], rows_v[p], osem[p])

    fire_idx(0, 0)
    fire_gathers(0, 0)
    fire_idx(1, 1)
    fire_gathers(1, 1)

    @pl.loop(0, NCHUNK, step=2)
    def _body(c):
        compute(c, 0)
        fire_idx(lax.rem(c + 2, NCHUNK), 0)
        compute(c + 1, 1)
        fire_idx(lax.rem(c + 3, NCHUNK), 1)
        drain_out(0)
        fire_gathers(lax.rem(c + 2, NCHUNK), 0)
        drain_out(1)
        fire_gathers(lax.rem(c + 3, NCHUNK), 1)

    drain(out_hbm.at[pl.ds(0, CHUNK)], rows_v[0], gsem[0])
    drain(out_hbm.at[pl.ds(0, CHUNK)], add_v[0], gsem[0])
    drain(out_hbm.at[pl.ds(0, CHUNK)], rows_v[1], gsem[1])
    drain(out_hbm.at[pl.ds(0, CHUNK)], add_v[1], gsem[1])


def kernel(token_ids, segment_ids, W_word, W_pos, W_seg, W_type, gamma, beta):
    del segment_ids
    comb = (W_pos[:S, None, :] + W_seg[0][None, None, :]
            + W_type[None, :, :]).reshape(S * TYPE_V, HID)
    pos6 = (TYPE_V * (jnp.arange(PERIOD * L, dtype=_i32) % S)).reshape(PERIOD, L)
    tok1d = token_ids.reshape(B * S)
    out = _emb_ln(tok1d, W_word, comb, pos6, gamma, beta)
    return out.reshape(B, S, HID)
